# baseline (speedup 1.0000x reference)
import functools

import jax
import jax.numpy as jnp
from jax import lax
from jax.experimental import pallas as pl
from jax.experimental.pallas import tpu as pltpu
from jax.experimental.pallas import tpu_sc as plsc

N_NODES_C = 10000
N_NODES_PAD = 10240
D_HALF = 128
N_TRIP = 160000
N_TRIP_PAD = 163840
N_SUBCORES = 16
TRIP_PER_SUB = N_TRIP_PAD // N_SUBCORES
CHUNK = 64
N_CHUNKS = TRIP_PER_SUB // CHUNK
ROWS_PER_SUB = N_NODES_PAD // N_SUBCORES
NSLOT = 5
PACK_W = 3 * CHUNK


def _body(x2, pack, out, idxb, iic, msg, acc, sem_i, sem_g, sem_s):
    c = lax.axis_index("c")
    s = lax.axis_index("s")

    def zero_row(t, _):
        for m in range(D_HALF // 16):
            msg[0][t, pl.ds(m * 16, 16)] = jnp.zeros((16,), jnp.float32)
        return 0

    lax.fori_loop(0, CHUNK, zero_row, 0)
    base = s * ROWS_PER_SUB
    for b in range(ROWS_PER_SUB // CHUNK):
        pltpu.sync_copy(msg[0], acc.at[pl.ds(base + b * CHUNK, CHUNK)])
    plsc.subcore_barrier()

    pbase = (c * N_SUBCORES + s) * (N_CHUNKS * PACK_W)

    def st_il(t, p):
        pltpu.async_copy(pack.at[pl.ds(pbase + t * PACK_W, PACK_W)],
                         idxb[p], sem_i[p])

    def st_g1(p):
        pltpu.make_async_copy(pack.at[pl.ds(0, PACK_W)], idxb[p],
                              sem_i[p]).wait()
        for m in range(CHUNK // 16):
            sl = pl.ds(m * 16, 16)
            iic[p][sl] = idxb[p][sl]
        pltpu.async_copy(x2.at[idxb[p].at[pl.ds(CHUNK, CHUNK)]],
                         msg[p], sem_g[p])

    def st_g2(p):
        pltpu.make_async_copy(x2.at[idxb[p].at[pl.ds(CHUNK, CHUNK)]],
                              msg[p], sem_g[p]).wait()
        pltpu.async_copy(x2.at[idxb[p].at[pl.ds(2 * CHUNK, CHUNK)]],
                         msg[p], sem_g[p], add=True)

    def st_sc(p):
        pltpu.make_async_copy(x2.at[idxb[p].at[pl.ds(CHUNK, CHUNK)]],
                              msg[p], sem_g[p]).wait()
        pltpu.async_copy(msg[p], acc.at[iic[p]], sem_s[p], add=True)

    def st_ws(p):
        pltpu.make_async_copy(msg[p], acc.at[iic[p]], sem_s[p]).wait()

    def group(i, _):
        t0 = NSLOT * i
        for u in range(NSLOT):
            t = t0 + u
            pl.when(jnp.logical_and(t >= NSLOT, t <= N_CHUNKS + NSLOT - 1))(
                lambda u=u: st_ws(u))
            pl.when(t <= N_CHUNKS - 1)(lambda t=t, u=u: st_il(t, u))
            pl.when(jnp.logical_and(t >= 1, t <= N_CHUNKS))(
                lambda u=u: st_g1((u - 1) % NSLOT))
            pl.when(jnp.logical_and(t >= 2, t <= N_CHUNKS + 1))(
                lambda u=u: st_g2((u - 2) % NSLOT))
            pl.when(jnp.logical_and(t >= 3, t <= N_CHUNKS + 2))(
                lambda u=u: st_sc((u - 3) % NSLOT))
        return 0

    n_iter = N_CHUNKS + NSLOT
    lax.fori_loop(0, n_iter // NSLOT, group, 0)
    plsc.subcore_barrier()

    pltpu.sync_copy(
        acc.at[pl.ds(base, ROWS_PER_SUB)],
        out.at[pl.ds(c * N_NODES_PAD + base, ROWS_PER_SUB)],
    )


@jax.jit
def _run(x2, pack):
    mesh = plsc.VectorSubcoreMesh(core_axis_name="c", subcore_axis_name="s")
    f = pl.kernel(
        _body,
        out_type=jax.ShapeDtypeStruct((2 * N_NODES_PAD, D_HALF), jnp.float32),
        mesh=mesh,
        scratch_types=[
            [pltpu.VMEM((PACK_W,), jnp.int32)] * NSLOT,
            [pltpu.VMEM((CHUNK,), jnp.int32)] * NSLOT,
            [pltpu.VMEM((CHUNK, D_HALF), jnp.float32)] * NSLOT,
            pltpu.VMEM_SHARED((N_NODES_PAD, D_HALF), jnp.float32),
            [pltpu.SemaphoreType.DMA] * NSLOT,
            [pltpu.SemaphoreType.DMA] * NSLOT,
            [pltpu.SemaphoreType.DMA] * NSLOT,
        ],
    )
    return f(x2, pack)


def kernel(x, a2_indices, e2, a3_indices, e3):
    x2 = x.reshape(2 * N_NODES_C, D_HALF)
    pad = N_TRIP_PAD - N_TRIP
    ai = jnp.concatenate([a3_indices[0], jnp.full((pad,), N_NODES_C, jnp.int32)])
    aj = jnp.concatenate([a3_indices[1], jnp.zeros((pad,), jnp.int32)])
    ak = jnp.concatenate([a3_indices[2], jnp.zeros((pad,), jnp.int32)])
    ii_r = ai.reshape(N_SUBCORES, N_CHUNKS, 1, CHUNK)
    packs = []
    for core in (0, 1):
        jj = (2 * aj + core).reshape(N_SUBCORES, N_CHUNKS, 1, CHUNK)
        kk = (2 * ak + core).reshape(N_SUBCORES, N_CHUNKS, 1, CHUNK)
        packs.append(jnp.concatenate([ii_r, jj, kk], axis=2).reshape(-1))
    pack = jnp.concatenate(packs)
    out = _run(x2, pack)
    return jnp.concatenate(
        [out[:N_NODES_C], out[N_NODES_PAD:N_NODES_PAD + N_NODES_C]], axis=1
    )
